# Initial kernel scaffold; baseline (speedup 1.0000x reference)
import jax
import jax.numpy as jnp
from jax.experimental import pallas as pl


def kernel(indices, tables, W_enc, b_enc, W_dec, b_dec):
    raise NotImplementedError("write your pallas kernel here")


# speedup vs baseline: 7.6828x; 7.6828x over previous
import functools

import jax
import jax.numpy as jnp
from jax import lax
from jax.experimental import pallas as pl
from jax.experimental.pallas import tpu as pltpu
from jax.experimental.pallas import tpu_sc as plsc

_NC = 2
_NS = 16
_NW = _NC * _NS

_DMA_ROWS = 128
_GROUP = 8


@functools.lru_cache(maxsize=None)
def _make_gather(rows_total: int, vt: int, d: int):
    assert rows_total % (_NW * _DMA_ROWS) == 0
    per_w = rows_total // _NW
    steps = per_w // _DMA_ROWS
    assert steps % _GROUP == 0
    groups = steps // _GROUP
    grp_rows = _GROUP * _DMA_ROWS

    mesh = plsc.VectorSubcoreMesh(core_axis_name="c", subcore_axis_name="s")

    @functools.partial(
        pl.kernel,
        mesh=mesh,
        out_type=jax.ShapeDtypeStruct((rows_total, d), jnp.float32),
        scratch_types=[
            pltpu.VMEM((steps, _DMA_ROWS), jnp.int32),
            pltpu.VMEM((grp_rows, d), jnp.float32),
            pltpu.SemaphoreType.DMA,
        ],
        compiler_params=pltpu.CompilerParams(use_tc_tiling_on_sc=False),
    )
    def gather_kernel(table_hbm, idx_hbm, out_hbm, idx_v, rows_v, sem):
        wid = lax.axis_index("s") * _NC + lax.axis_index("c")
        base = wid * per_w
        pltpu.sync_copy(idx_hbm.at[wid], idx_v)

        def body(g, carry):
            copies = []
            for j in range(_GROUP):
                cp = pltpu.async_copy(
                    table_hbm.at[idx_v.at[g * _GROUP + j]],
                    rows_v.at[pl.ds(j * _DMA_ROWS, _DMA_ROWS)],
                    sem,
                )
                copies.append(cp)
            for cp in copies:
                cp.wait()
            pltpu.sync_copy(
                rows_v, out_hbm.at[pl.ds(base + g * grp_rows, grp_rows)]
            )
            return carry

        lax.fori_loop(0, groups, body, 0)

    return gather_kernel


def _mlp_body(x_ref, we_ref, be_ref, wd_ref, bd_ref, o_ref):
    x = x_ref[...]
    z = jnp.dot(x, we_ref[...], preferred_element_type=jnp.float32)
    z = jnp.maximum(z + be_ref[...], 0.0)
    y = jnp.dot(z, wd_ref[...], preferred_element_type=jnp.float32)
    y = y + bd_ref[...]
    o_ref[...] = 1.0 / (1.0 + jnp.exp(-y))


@functools.lru_cache(maxsize=None)
def _make_mlp(b: int, out_dim: int, latent: int, bm: int):
    grid = (b // bm,)
    return pl.pallas_call(
        _mlp_body,
        grid=grid,
        in_specs=[
            pl.BlockSpec((bm, out_dim), lambda i: (i, 0)),
            pl.BlockSpec((out_dim, latent), lambda i: (0, 0)),
            pl.BlockSpec((1, latent), lambda i: (0, 0)),
            pl.BlockSpec((latent, out_dim), lambda i: (0, 0)),
            pl.BlockSpec((1, out_dim), lambda i: (0, 0)),
        ],
        out_specs=pl.BlockSpec((bm, out_dim), lambda i: (i, 0)),
        out_shape=jax.ShapeDtypeStruct((b, out_dim), jnp.float32),
    )


def kernel(indices, tables, W_enc, b_enc, W_dec, b_dec):
    b, f = indices.shape
    _, v, d = tables.shape
    out_dim, latent = W_enc.shape

    flat_idx = indices.astype(jnp.int32) + (
        jnp.arange(f, dtype=jnp.int32) * v
    )[None, :]
    rows_total = b * f
    per_w = rows_total // _NW
    idx3d = flat_idx.reshape(_NW, per_w // _DMA_ROWS, _DMA_ROWS)
    table_flat = tables.reshape(f * v, d)

    gathered = _make_gather(rows_total, f * v, d)(table_flat, idx3d)
    x = gathered.reshape(b, f * d)

    mlp = _make_mlp(b, out_dim, latent, 2048)
    return mlp(x, W_enc, b_enc.reshape(1, latent), W_dec, b_dec.reshape(1, out_dim))


# speedup vs baseline: 35.9974x; 4.6855x over previous
import functools

import jax
import jax.numpy as jnp
from jax import lax
from jax.experimental import pallas as pl
from jax.experimental.pallas import tpu as pltpu
from jax.experimental.pallas import tpu_sc as plsc

_NC = 2
_NS = 16
_NW = _NC * _NS

_CHUNK = 2048
_UNROLL = 4


@functools.lru_cache(maxsize=None)
def _make_gather_t(nrows: int, v: int, b: int, d: int):
    assert nrows % _NW == 0
    per_w = nrows // _NW
    nchunks = b // _CHUNK
    inner = _CHUNK // (16 * _UNROLL)
    dshift = d.bit_length() - 1
    assert 1 << dshift == d

    mesh = plsc.VectorSubcoreMesh(core_axis_name="c", subcore_axis_name="s")

    @functools.partial(
        pl.kernel,
        mesh=mesh,
        out_type=jax.ShapeDtypeStruct((nrows, b), jnp.float32),
        scratch_types=[
            pltpu.VMEM((v,), jnp.float32),
            pltpu.VMEM((_CHUNK,), jnp.int32),
            pltpu.VMEM((_CHUNK,), jnp.float32),
        ],
        compiler_params=pltpu.CompilerParams(
            use_tc_tiling_on_sc=True, needs_layout_passes=False
        ),
    )
    def gather_kernel(tbl_hbm, idx_hbm, out_hbm, plane_v, idx_v, out_v):
        wid = lax.axis_index("s") * _NC + lax.axis_index("c")

        def row_body(i, c0):
            r = wid * per_w + i
            f = lax.shift_right_logical(r, dshift)
            pltpu.sync_copy(tbl_hbm.at[r, :], plane_v)

            def chunk_body(c, c1):
                pltpu.sync_copy(
                    idx_hbm.at[pl.ds(f * b + c * _CHUNK, _CHUNK)], idx_v
                )

                def vec_body(t, c2):
                    base = t * (16 * _UNROLL)
                    for u in range(_UNROLL):
                        ii = idx_v[pl.ds(base + u * 16, 16)]
                        out_v[pl.ds(base + u * 16, 16)] = plsc.load_gather(
                            plane_v, [ii]
                        )
                    return c2

                lax.fori_loop(0, inner, vec_body, 0)
                pltpu.sync_copy(out_v, out_hbm.at[r, pl.ds(c * _CHUNK, _CHUNK)])
                return c1

            lax.fori_loop(0, nchunks, chunk_body, 0)
            return c0

        lax.fori_loop(0, per_w, row_body, 0)

    return gather_kernel


def _mlp_t_body(xt_ref, we_ref, be_ref, wd_ref, bd_ref, o_ref):
    xt = xt_ref[...]
    z = lax.dot_general(
        we_ref[...], xt, (((0,), (0,)), ((), ())),
        preferred_element_type=jnp.float32,
    )
    z = jnp.maximum(z + be_ref[...], 0.0)
    y = lax.dot_general(
        wd_ref[...], z, (((0,), (0,)), ((), ())),
        preferred_element_type=jnp.float32,
    )
    y = y + bd_ref[...]
    o_ref[...] = 1.0 / (1.0 + jnp.exp(-y))


@functools.lru_cache(maxsize=None)
def _make_mlp_t(b: int, out_dim: int, latent: int, bn: int):
    grid = (b // bn,)
    return pl.pallas_call(
        _mlp_t_body,
        grid=grid,
        in_specs=[
            pl.BlockSpec((out_dim, bn), lambda i: (0, i)),
            pl.BlockSpec((out_dim, latent), lambda i: (0, 0)),
            pl.BlockSpec((latent, 1), lambda i: (0, 0)),
            pl.BlockSpec((latent, out_dim), lambda i: (0, 0)),
            pl.BlockSpec((out_dim, 1), lambda i: (0, 0)),
        ],
        out_specs=pl.BlockSpec((out_dim, bn), lambda i: (0, i)),
        out_shape=jax.ShapeDtypeStruct((out_dim, b), jnp.float32),
    )


def kernel(indices, tables, W_enc, b_enc, W_dec, b_dec):
    b, f = indices.shape
    _, v, d = tables.shape
    out_dim, latent = W_enc.shape

    planes = tables.transpose(0, 2, 1).reshape(f * d, v)
    idx_flat = indices.astype(jnp.int32).T.reshape(-1)

    xt = _make_gather_t(f * d, v, b, d)(planes, idx_flat)

    mlp = _make_mlp_t(b, out_dim, latent, 2048)
    out_t = mlp(
        xt, W_enc, b_enc.reshape(latent, 1), W_dec, b_dec.reshape(out_dim, 1)
    )
    return out_t.T


# speedup vs baseline: 50.9795x; 1.4162x over previous
import functools

import jax
import jax.numpy as jnp
from jax import lax
from jax.experimental import pallas as pl
from jax.experimental.pallas import tpu as pltpu
from jax.experimental.pallas import tpu_sc as plsc

_NC = 2
_NS = 16
_NW = _NC * _NS

_CHUNK = 2048
_UNROLL = 4


@functools.lru_cache(maxsize=None)
def _make_gather_t(nrows: int, v: int, b: int, d: int):
    assert nrows % _NW == 0
    per_w = nrows // _NW
    nchunks = b // _CHUNK
    inner = _CHUNK // (16 * _UNROLL)
    dshift = d.bit_length() - 1
    assert 1 << dshift == d

    mesh = plsc.VectorSubcoreMesh(core_axis_name="c", subcore_axis_name="s")

    @functools.partial(
        pl.kernel,
        mesh=mesh,
        out_type=jax.ShapeDtypeStruct((nrows, b), jnp.float32),
        scratch_types=[
            pltpu.VMEM((v,), jnp.float32),
            pltpu.VMEM((_CHUNK,), jnp.int32),
            pltpu.VMEM((_CHUNK,), jnp.int32),
            pltpu.VMEM((_CHUNK,), jnp.float32),
            pltpu.VMEM((_CHUNK,), jnp.float32),
            pltpu.SemaphoreType.DMA,
            pltpu.SemaphoreType.DMA,
            pltpu.SemaphoreType.DMA,
            pltpu.SemaphoreType.DMA,
        ],
        compiler_params=pltpu.CompilerParams(
            use_tc_tiling_on_sc=True, needs_layout_passes=False
        ),
    )
    def gather_kernel(
        tbl_hbm, idx_hbm, out_hbm,
        plane_v, idx_a, idx_b, out_a, out_b, si_a, si_b, so_a, so_b,
    ):
        wid = lax.axis_index("s") * _NC + lax.axis_index("c")
        idx_bufs = (idx_a, idx_b)
        out_bufs = (out_a, out_b)
        si = (si_a, si_b)
        so = (so_a, so_b)

        def row_body(i, c0):
            r = wid * per_w + i
            f = lax.shift_right_logical(r, dshift)
            ibase = f * b
            pltpu.async_copy(
                idx_hbm.at[pl.ds(ibase, _CHUNK)], idx_a, si_a
            )
            pltpu.async_copy(
                idx_hbm.at[pl.ds(ibase + _CHUNK, _CHUNK)], idx_b, si_b
            )
            pltpu.sync_copy(tbl_hbm.at[r, :], plane_v)

            for c in range(nchunks):
                u = c % 2
                if c >= 2:
                    pltpu.make_async_copy(
                        out_bufs[u],
                        out_hbm.at[r, pl.ds((c - 2) * _CHUNK, _CHUNK)],
                        so[u],
                    ).wait()
                pltpu.make_async_copy(
                    idx_hbm.at[pl.ds(ibase + c * _CHUNK, _CHUNK)],
                    idx_bufs[u],
                    si[u],
                ).wait()

                def vec_body(t, c2, _iv=idx_bufs[u], _ov=out_bufs[u]):
                    base = t * (16 * _UNROLL)
                    for uu in range(_UNROLL):
                        ii = _iv[pl.ds(base + uu * 16, 16)]
                        _ov[pl.ds(base + uu * 16, 16)] = plsc.load_gather(
                            plane_v, [ii]
                        )
                    return c2

                lax.fori_loop(0, inner, vec_body, 0)
                pltpu.async_copy(
                    out_bufs[u], out_hbm.at[r, pl.ds(c * _CHUNK, _CHUNK)], so[u]
                )
                if c + 2 < nchunks:
                    pltpu.async_copy(
                        idx_hbm.at[pl.ds(ibase + (c + 2) * _CHUNK, _CHUNK)],
                        idx_bufs[u],
                        si[u],
                    )

            for c in (nchunks - 2, nchunks - 1):
                u = c % 2
                pltpu.make_async_copy(
                    out_bufs[u], out_hbm.at[r, pl.ds(c * _CHUNK, _CHUNK)], so[u]
                ).wait()
            return c0

        lax.fori_loop(0, per_w, row_body, 0)

    return gather_kernel


def _mlp_t_body(xt_ref, we_ref, be_ref, wd_ref, bd_ref, o_ref):
    xt = xt_ref[...]
    z = lax.dot_general(
        we_ref[...], xt, (((0,), (0,)), ((), ())),
        preferred_element_type=jnp.float32,
    )
    z = jnp.maximum(z + be_ref[...], 0.0)
    y = lax.dot_general(
        wd_ref[...], z, (((0,), (0,)), ((), ())),
        preferred_element_type=jnp.float32,
    )
    y = y + bd_ref[...]
    o_ref[...] = 1.0 / (1.0 + jnp.exp(-y))


@functools.lru_cache(maxsize=None)
def _make_mlp_t(b: int, out_dim: int, latent: int, bn: int):
    grid = (b // bn,)
    return pl.pallas_call(
        _mlp_t_body,
        grid=grid,
        in_specs=[
            pl.BlockSpec((out_dim, bn), lambda i: (0, i)),
            pl.BlockSpec((out_dim, latent), lambda i: (0, 0)),
            pl.BlockSpec((latent, 1), lambda i: (0, 0)),
            pl.BlockSpec((latent, out_dim), lambda i: (0, 0)),
            pl.BlockSpec((out_dim, 1), lambda i: (0, 0)),
        ],
        out_specs=pl.BlockSpec((out_dim, bn), lambda i: (0, i)),
        out_shape=jax.ShapeDtypeStruct((out_dim, b), jnp.float32),
    )


def kernel(indices, tables, W_enc, b_enc, W_dec, b_dec):
    b, f = indices.shape
    _, v, d = tables.shape
    out_dim, latent = W_enc.shape

    planes = tables.transpose(0, 2, 1).reshape(f * d, v)
    idx_flat = indices.astype(jnp.int32).T.reshape(-1)

    xt = _make_gather_t(f * d, v, b, d)(planes, idx_flat)

    mlp = _make_mlp_t(b, out_dim, latent, 2048)
    out_t = mlp(
        xt, W_enc, b_enc.reshape(latent, 1), W_dec, b_dec.reshape(out_dim, 1)
    )
    return out_t.T
